# Initial kernel scaffold; baseline (speedup 1.0000x reference)
import jax
import jax.numpy as jnp
from jax.experimental import pallas as pl


def kernel(node_feat, edge_index, W1, W2, W3, W4, conv1_w, conv1_b, conv2_w, conv2_b, dense_w, dense_b, out_w, out_b):
    raise NotImplementedError("write your pallas kernel here")


# speedup vs baseline: 8.9534x; 8.9534x over previous
import functools

import jax
import jax.numpy as jnp
from jax import lax
from jax.experimental import pallas as pl
from jax.experimental.pallas import tpu as pltpu
from jax.experimental.pallas import tpu_sc as plsc

N = 10000
E = 320000
B = 20
NPG = 500
D_IN = 128
C = 32
TOT = 97
C1 = 16
C2 = 32
KS2 = 5
P2 = 246
NH = 128

NW = 32
EPW = E // NW
CH = 80
NCHUNK = EPW // CH
RPS = 624
RTAIL = N - RPS * 16


def _mm_body(x_ref, w_ref, o_ref):
    o_ref[...] = jnp.dot(x_ref[...], w_ref[...],
                         preferred_element_type=jnp.float32)


def _input_proj(nf, w1):
    blk = 2000
    return pl.pallas_call(
        _mm_body,
        grid=(N // blk,),
        in_specs=[pl.BlockSpec((blk, D_IN), lambda i: (i, 0)),
                  pl.BlockSpec((D_IN, C), lambda i: (0, 0))],
        out_specs=pl.BlockSpec((blk, C), lambda i: (i, 0)),
        out_shape=jax.ShapeDtypeStruct((N, C), jnp.float32),
    )(nf, w1)


def _comb1_body(y_ref, p_ref, dp_ref, w_ref, h_ref, y2_ref, deg_ref):
    deg = dp_ref[0] + dp_ref[1] + 1.0
    h = jnp.tanh((y_ref[...] + p_ref[0] + p_ref[1]) / deg)
    h_ref[...] = h
    y2_ref[...] = jnp.dot(h, w_ref[...], preferred_element_type=jnp.float32)
    deg_ref[...] = deg


def _comb1(y, parts, dparts, w_next):
    blk = 2000
    return pl.pallas_call(
        _comb1_body,
        grid=(N // blk,),
        in_specs=[pl.BlockSpec((blk, C), lambda i: (i, 0)),
                  pl.BlockSpec((2, blk, C), lambda i: (0, i, 0)),
                  pl.BlockSpec((2, blk, 1), lambda i: (0, i, 0)),
                  pl.BlockSpec((C, C), lambda i: (0, 0))],
        out_specs=[pl.BlockSpec((blk, C), lambda i: (i, 0)),
                   pl.BlockSpec((blk, C), lambda i: (i, 0)),
                   pl.BlockSpec((blk, 1), lambda i: (i, 0))],
        out_shape=[jax.ShapeDtypeStruct((N, C), jnp.float32),
                   jax.ShapeDtypeStruct((N, C), jnp.float32),
                   jax.ShapeDtypeStruct((N, 1), jnp.float32)],
    )(y, parts, dparts, w_next)


def _combn_body(y_ref, p_ref, deg_ref, w_ref, h_ref, y2_ref):
    h = jnp.tanh((y_ref[...] + p_ref[0] + p_ref[1]) / deg_ref[...])
    h_ref[...] = h
    y2_ref[...] = jnp.dot(h, w_ref[...], preferred_element_type=jnp.float32)


def _combn(y, parts, deg, w_next):
    blk = 2000
    cout = w_next.shape[1]
    return pl.pallas_call(
        _combn_body,
        grid=(N // blk,),
        in_specs=[pl.BlockSpec((blk, C), lambda i: (i, 0)),
                  pl.BlockSpec((2, blk, C), lambda i: (0, i, 0)),
                  pl.BlockSpec((blk, 1), lambda i: (i, 0)),
                  pl.BlockSpec((C, cout), lambda i: (0, 0))],
        out_specs=[pl.BlockSpec((blk, C), lambda i: (i, 0)),
                   pl.BlockSpec((blk, cout), lambda i: (i, 0))],
        out_shape=[jax.ShapeDtypeStruct((N, C), jnp.float32),
                   jax.ShapeDtypeStruct((N, cout), jnp.float32)],
    )(y, parts, deg, w_next)


def _head_body(h1_ref, h2_ref, h3_ref, y4_ref, pa_ref, pb_ref, deg_ref,
               c1w_ref, c1b_ref, w2r_ref, c2b_ref, r2_ref):
    y4 = y4_ref[0]
    h4 = jnp.tanh((y4 + pa_ref[0] + pb_ref[0]) / deg_ref[0])
    g = jnp.concatenate([h1_ref[0], h2_ref[0], h3_ref[0], h4], axis=1)
    key = h4
    iden = (lax.broadcasted_iota(jnp.int32, (NPG, NPG), 0)
            == lax.broadcasted_iota(jnp.int32, (NPG, NPG), 1)
            ).astype(jnp.float32)
    krow = lax.dot_general(key, iden, (((0,), (0,)), ((), ())),
                           preferred_element_type=jnp.float32)
    ii = lax.broadcasted_iota(jnp.int32, (NPG, NPG), 0)
    jj = lax.broadcasted_iota(jnp.int32, (NPG, NPG), 1)
    gt = (key > krow).astype(jnp.float32)
    tie = jnp.logical_and(key == krow, ii < jj).astype(jnp.float32)
    ranki = jnp.sum(gt + tie, axis=0, keepdims=True).astype(jnp.int32)
    c1 = jnp.dot(g, c1w_ref[...], preferred_element_type=jnp.float32)
    pe = 2 * lax.broadcasted_iota(jnp.int32, (NPG // 2, NPG), 0)
    m_even = (pe == ranki).astype(jnp.float32)
    m_odd = (pe + 1 == ranki).astype(jnp.float32)
    a_even = jnp.dot(m_even, c1, preferred_element_type=jnp.float32)
    a_odd = jnp.dot(m_odd, c1, preferred_element_type=jnp.float32)
    pooled = jnp.maximum(jnp.maximum(a_even, a_odd) + c1b_ref[...], 0.0)
    wnd = jnp.concatenate([pooled[t:t + P2, :] for t in range(KS2)], axis=1)
    r2 = jnp.dot(wnd, w2r_ref[...], preferred_element_type=jnp.float32)
    r2_ref[0] = jnp.maximum(r2 + c2b_ref[...], 0.0)


def _head(h1r, h2r, h3r, y4r, p4a, p4b, degr, c1w, c1b2, w2r, c2b2):
    spec_nc = pl.BlockSpec((1, NPG, C), lambda b: (b, 0, 0))
    spec_n1 = pl.BlockSpec((1, NPG, 1), lambda b: (b, 0, 0))
    full = lambda shape: pl.BlockSpec(shape, lambda b: tuple(0 for _ in shape))
    return pl.pallas_call(
        _head_body,
        grid=(B,),
        in_specs=[spec_nc, spec_nc, spec_nc, spec_n1, spec_n1, spec_n1,
                  spec_n1, full((TOT, C1)), full((1, C1)),
                  full((C1 * KS2, C2)), full((1, C2))],
        out_specs=pl.BlockSpec((1, P2, C2), lambda b: (b, 0, 0)),
        out_shape=jax.ShapeDtypeStruct((B, P2, C2), jnp.float32),
    )(h1r, h2r, h3r, y4r, p4a, p4b, degr, c1w, c1b2, w2r, c2b2)


def _final_body(x_ref, dw_ref, db_ref, ow_ref, ob_ref, o_ref):
    hdn = jnp.maximum(
        jnp.dot(x_ref[...], dw_ref[...], preferred_element_type=jnp.float32)
        + db_ref[...], 0.0)
    logits = jnp.dot(hdn, ow_ref[...],
                     preferred_element_type=jnp.float32) + ob_ref[...]
    m = jnp.max(logits, axis=1, keepdims=True)
    e = logits - m
    lse = jnp.log(jnp.sum(jnp.exp(e), axis=1, keepdims=True))
    o_ref[...] = e - lse


def _final(flat, dwr, db2, ow, ob2):
    full = lambda shape: pl.BlockSpec(shape, lambda: tuple(0 for _ in shape))
    return pl.pallas_call(
        _final_body,
        in_specs=[full((B, P2 * C2)), full((P2 * C2, NH)), full((1, NH)),
                  full((NH, 2)), full((1, 2))],
        out_specs=full((B, 2)),
        out_shape=jax.ShapeDtypeStruct((B, 2), jnp.float32),
    )(flat, dwr, db2, ow, ob2)


@functools.cache
def _mesh():
    return plsc.VectorSubcoreMesh(core_axis_name="c", subcore_axis_name="s")


def _edge_rows(y, srcr, dstr, zrows, zcol):
    @functools.partial(
        pl.kernel,
        out_type=[jax.ShapeDtypeStruct((2 * N, C), jnp.float32),
                  jax.ShapeDtypeStruct((2 * N,), jnp.float32)],
        mesh=_mesh(),
        compiler_params=pltpu.CompilerParams(use_tc_tiling_on_sc=False),
        scratch_types=[
            pltpu.VMEM((NCHUNK, CH), jnp.int32),
            pltpu.VMEM((NCHUNK, CH), jnp.int32),
            pltpu.VMEM((CH, C), jnp.float32),
            pltpu.VMEM((CH,), jnp.float32),
            pltpu.VMEM((RPS, C), jnp.float32),
            pltpu.VMEM((RPS,), jnp.float32),
            pltpu.VMEM_SHARED((N, C), jnp.float32),
            pltpu.VMEM_SHARED((N,), jnp.float32),
            pltpu.SemaphoreType.DMA,
        ],
    )
    def k(y_hbm, src_hbm, dst_hbm, zr_hbm, zc_hbm, out_hbm, dout_hbm,
          srcv, dstv, rows, ones, stg, stgc, acc, dacc, sem):
        c = lax.axis_index("c")
        s = lax.axis_index("s")
        wid = c * 16 + s
        r0 = s * RPS
        t0 = RPS * 16
        pltpu.sync_copy(zr_hbm.at[pl.ds(r0, RPS)], stg)
        pltpu.sync_copy(zc_hbm.at[pl.ds(r0, RPS)], stgc)
        pltpu.sync_copy(stg, acc.at[pl.ds(r0, RPS)])
        pltpu.sync_copy(stgc, dacc.at[pl.ds(r0, RPS)])

        @pl.when(s == 0)
        def _():
            pltpu.sync_copy(stg.at[pl.ds(0, RTAIL)],
                            acc.at[pl.ds(t0, RTAIL)])
            pltpu.sync_copy(stgc.at[pl.ds(0, RTAIL)],
                            dacc.at[pl.ds(t0, RTAIL)])

        for i in range(CH // 16):
            ones[pl.ds(i * 16, 16)] = jnp.ones((16,), jnp.float32)
        pltpu.sync_copy(src_hbm.at[wid], srcv)
        pltpu.sync_copy(dst_hbm.at[wid], dstv)
        plsc.subcore_barrier()

        def body(i, carry):
            pltpu.async_copy(y_hbm.at[srcv.at[i]], rows, sem).wait()
            pltpu.sync_copy(rows, acc.at[dstv.at[i]], add=True)
            pltpu.sync_copy(ones, dacc.at[dstv.at[i]], add=True)
            return carry

        lax.fori_loop(0, NCHUNK, body, 0)
        plsc.subcore_barrier()
        base = c * N + r0
        pltpu.sync_copy(acc.at[pl.ds(r0, RPS)], stg)
        pltpu.sync_copy(dacc.at[pl.ds(r0, RPS)], stgc)
        pltpu.sync_copy(stg, out_hbm.at[pl.ds(base, RPS)])
        pltpu.sync_copy(stgc, dout_hbm.at[pl.ds(base, RPS)])

        @pl.when(s == 0)
        def _():
            pltpu.sync_copy(acc.at[pl.ds(t0, RTAIL)],
                            stg.at[pl.ds(0, RTAIL)])
            pltpu.sync_copy(dacc.at[pl.ds(t0, RTAIL)],
                            stgc.at[pl.ds(0, RTAIL)])
            pltpu.sync_copy(stg.at[pl.ds(0, RTAIL)],
                            out_hbm.at[pl.ds(c * N + t0, RTAIL)])
            pltpu.sync_copy(stgc.at[pl.ds(0, RTAIL)],
                            dout_hbm.at[pl.ds(c * N + t0, RTAIL)])

    return k(y, srcr, dstr, zrows, zcol)


def _edge_rows_nodeg(y, srcr, dstr, zrows):
    @functools.partial(
        pl.kernel,
        out_type=jax.ShapeDtypeStruct((2 * N, C), jnp.float32),
        mesh=_mesh(),
        compiler_params=pltpu.CompilerParams(use_tc_tiling_on_sc=False),
        scratch_types=[
            pltpu.VMEM((NCHUNK, CH), jnp.int32),
            pltpu.VMEM((NCHUNK, CH), jnp.int32),
            pltpu.VMEM((CH, C), jnp.float32),
            pltpu.VMEM((RPS, C), jnp.float32),
            pltpu.VMEM_SHARED((N, C), jnp.float32),
            pltpu.SemaphoreType.DMA,
        ],
    )
    def k(y_hbm, src_hbm, dst_hbm, zr_hbm, out_hbm, srcv, dstv, rows, stg,
          acc, sem):
        c = lax.axis_index("c")
        s = lax.axis_index("s")
        wid = c * 16 + s
        r0 = s * RPS
        t0 = RPS * 16
        pltpu.sync_copy(zr_hbm.at[pl.ds(r0, RPS)], stg)
        pltpu.sync_copy(stg, acc.at[pl.ds(r0, RPS)])

        @pl.when(s == 0)
        def _():
            pltpu.sync_copy(stg.at[pl.ds(0, RTAIL)],
                            acc.at[pl.ds(t0, RTAIL)])

        pltpu.sync_copy(src_hbm.at[wid], srcv)
        pltpu.sync_copy(dst_hbm.at[wid], dstv)
        plsc.subcore_barrier()

        def body(i, carry):
            pltpu.async_copy(y_hbm.at[srcv.at[i]], rows, sem).wait()
            pltpu.sync_copy(rows, acc.at[dstv.at[i]], add=True)
            return carry

        lax.fori_loop(0, NCHUNK, body, 0)
        plsc.subcore_barrier()
        base = c * N + r0
        pltpu.sync_copy(acc.at[pl.ds(r0, RPS)], stg)
        pltpu.sync_copy(stg, out_hbm.at[pl.ds(base, RPS)])

        @pl.when(s == 0)
        def _():
            pltpu.sync_copy(acc.at[pl.ds(t0, RTAIL)],
                            stg.at[pl.ds(0, RTAIL)])
            pltpu.sync_copy(stg.at[pl.ds(0, RTAIL)],
                            out_hbm.at[pl.ds(c * N + t0, RTAIL)])

    return k(y, srcr, dstr, zrows)


def _edge_scalar(yflat, srcr, dstr, zcol):
    @functools.partial(
        pl.kernel,
        out_type=jax.ShapeDtypeStruct((2 * N,), jnp.float32),
        mesh=_mesh(),
        compiler_params=pltpu.CompilerParams(use_tc_tiling_on_sc=False),
        scratch_types=[
            pltpu.VMEM((NCHUNK, CH), jnp.int32),
            pltpu.VMEM((NCHUNK, CH), jnp.int32),
            pltpu.VMEM((CH,), jnp.float32),
            pltpu.VMEM((RPS,), jnp.float32),
            pltpu.VMEM_SHARED((N,), jnp.float32),
            pltpu.SemaphoreType.DMA,
        ],
    )
    def k(y_hbm, src_hbm, dst_hbm, zc_hbm, out_hbm, srcv, dstv, vals, stgc,
          dacc, sem):
        c = lax.axis_index("c")
        s = lax.axis_index("s")
        wid = c * 16 + s
        r0 = s * RPS
        t0 = RPS * 16
        pltpu.sync_copy(zc_hbm.at[pl.ds(r0, RPS)], stgc)
        pltpu.sync_copy(stgc, dacc.at[pl.ds(r0, RPS)])

        @pl.when(s == 0)
        def _():
            pltpu.sync_copy(stgc.at[pl.ds(0, RTAIL)],
                            dacc.at[pl.ds(t0, RTAIL)])

        pltpu.sync_copy(src_hbm.at[wid], srcv)
        pltpu.sync_copy(dst_hbm.at[wid], dstv)
        plsc.subcore_barrier()

        def body(i, carry):
            pltpu.async_copy(y_hbm.at[srcv.at[i]], vals, sem).wait()
            pltpu.sync_copy(vals, dacc.at[dstv.at[i]], add=True)
            return carry

        lax.fori_loop(0, NCHUNK, body, 0)
        plsc.subcore_barrier()
        pltpu.sync_copy(dacc.at[pl.ds(r0, RPS)], stgc)
        pltpu.sync_copy(stgc, out_hbm.at[pl.ds(c * N + r0, RPS)])

        @pl.when(s == 0)
        def _():
            pltpu.sync_copy(dacc.at[pl.ds(t0, RTAIL)],
                            stgc.at[pl.ds(0, RTAIL)])
            pltpu.sync_copy(stgc.at[pl.ds(0, RTAIL)],
                            out_hbm.at[pl.ds(c * N + t0, RTAIL)])

    return k(yflat, srcr, dstr, zcol)


def kernel(node_feat, edge_index, W1, W2, W3, W4, conv1_w, conv1_b, conv2_w,
           conv2_b, dense_w, dense_b, out_w, out_b):
    src = edge_index[0].astype(jnp.int32).reshape(NW, NCHUNK, CH)
    dst = edge_index[1].astype(jnp.int32).reshape(NW, NCHUNK, CH)
    zrows = jnp.zeros((N, C), jnp.float32)
    zcol = jnp.zeros((N,), jnp.float32)

    y1 = _input_proj(node_feat, W1)
    p1, dp1 = _edge_rows(y1, src, dst, zrows, zcol)
    h1, y2, deg = _comb1(y1, p1.reshape(2, N, C), dp1.reshape(2, N, 1), W2)
    p2 = _edge_rows_nodeg(y2, src, dst, zrows)
    h2, y3 = _combn(y2, p2.reshape(2, N, C), deg, W3)
    p3 = _edge_rows_nodeg(y3, src, dst, zrows)
    h3, y4 = _combn(y3, p3.reshape(2, N, C), deg, W4)
    p4 = _edge_scalar(y4.reshape(N), src, dst, zcol)

    h1r = h1.reshape(B, NPG, C)
    h2r = h2.reshape(B, NPG, C)
    h3r = h3.reshape(B, NPG, C)
    y4r = y4.reshape(B, NPG, 1)
    p4a = p4[:N].reshape(B, NPG, 1)
    p4b = p4[N:].reshape(B, NPG, 1)
    degr = deg.reshape(B, NPG, 1)
    w2r = conv2_w.transpose(2, 1, 0).reshape(C1 * KS2, C2)
    r2 = _head(h1r, h2r, h3r, y4r, p4a, p4b, degr, conv1_w,
               conv1_b.reshape(1, C1), w2r, conv2_b.reshape(1, C2))

    flat = r2.reshape(B, P2 * C2)
    dwr = dense_w.reshape(C2, P2, NH).transpose(1, 0, 2).reshape(P2 * C2, NH)
    return _final(flat, dwr, dense_b.reshape(1, NH), out_w,
                  out_b.reshape(1, 2))
